# baseline (speedup 1.0000x reference)
import functools

import jax
import jax.numpy as jnp
from jax import lax
from jax.experimental import pallas as pl
from jax.experimental.pallas import tpu as pltpu
from jax.experimental.pallas import tpu_sc as plsc

A = 589824
EPS = 1e-7
NC = 2
NS = 16
NW = NC * NS
PER_W = A // NW
CHUNK = 4608
NCHUNK = PER_W // CHUNK
GROUPS = CHUNK // 16


def _sc_body(out_hbm, tgt_hbm, lbl_hbm, res_hbm,
             obuf0, tbuf0, lbuf0, obuf1, tbuf1, lbuf1, res_v,
             sem0, sem1):
    wid = lax.axis_index("s") * NC + lax.axis_index("c")
    a_base = wid * PER_W
    bufs = ((obuf0, tbuf0, lbuf0, sem0), (obuf1, tbuf1, lbuf1, sem1))

    def issue(c):
        ob, tb, lb, sem = bufs[c % 2]
        astart = a_base + c * CHUNK
        return (
            pltpu.async_copy(out_hbm.at[pl.ds(astart * 4, CHUNK * 4)], ob, sem),
            pltpu.async_copy(tgt_hbm.at[pl.ds(astart * 4, CHUNK * 4)], tb, sem),
            pltpu.async_copy(lbl_hbm.at[pl.ds(astart, CHUNK)], lb, sem),
        )

    pending = [issue(0), issue(1)]

    pat = lax.shift_right_logical(lax.iota(jnp.int32, 16), 2)
    dnums = lax.GatherDimensionNumbers(
        offset_dims=(), collapsed_slice_dims=(0,), start_index_map=(0,))

    def expand4(vec, g):
        idx = (pat + (4 * g)).reshape(16, 1)
        return lax.gather(vec, idx, dnums, (1,),
                          mode=lax.GatherScatterMode.PROMISE_IN_BOUNDS)

    zero = jnp.zeros((16,), jnp.float32)
    one = jnp.ones((16,), jnp.float32)
    a_acc, p_acc, v_acc = zero, zero, zero

    for c in range(NCHUNK):
        for d in pending[c]:
            d.wait()
        ob, tb, lb, _ = bufs[c % 2]

        def jbody(j, carry, ob=ob, tb=tb, lb=lb):
            aa, pa, va = carry
            lbl16 = lb[pl.ds(j * 16, 16)]
            m16 = jnp.where(lbl16 > 0.0, one, zero)
            pa = pa + m16
            va = va + jnp.where(lbl16 != -1.0, one, zero)
            for g in range(4):
                e0 = j * 64 + g * 16
                o = ob[pl.ds(e0, 16)]
                t = tb[pl.ds(e0, 16)]
                diff = jnp.abs(o - t)
                l = jnp.where(diff < 1.0, 0.5 * diff * diff, diff - 0.5)
                aa = aa + expand4(m16, g) * l
            return aa, pa, va

        a_acc, p_acc, v_acc = lax.fori_loop(
            0, GROUPS, jbody, (a_acc, p_acc, v_acc))

        nxt = c + 2
        if nxt < NCHUNK:
            pending.append(issue(nxt))

    res_v[pl.ds(0, 16)] = a_acc
    res_v[pl.ds(16, 16)] = p_acc
    res_v[pl.ds(32, 16)] = v_acc
    pltpu.sync_copy(res_v, res_hbm.at[wid])


@jax.jit
def _rpn_loss(out_flat, tgt_flat, lbl_flat):
    mesh = plsc.VectorSubcoreMesh(core_axis_name="c", subcore_axis_name="s")
    partials = pl.kernel(
        _sc_body,
        mesh=mesh,
        out_type=jax.ShapeDtypeStruct((NW, 48), jnp.float32),
        scratch_types=[
            pltpu.VMEM((CHUNK * 4,), jnp.float32),
            pltpu.VMEM((CHUNK * 4,), jnp.float32),
            pltpu.VMEM((CHUNK,), jnp.float32),
            pltpu.VMEM((CHUNK * 4,), jnp.float32),
            pltpu.VMEM((CHUNK * 4,), jnp.float32),
            pltpu.VMEM((CHUNK,), jnp.float32),
            pltpu.VMEM((48,), jnp.float32),
            pltpu.SemaphoreType.DMA,
            pltpu.SemaphoreType.DMA,
        ],
    )(out_flat, tgt_flat, lbl_flat)
    a = jnp.sum(partials[:, 0:16])
    pos = jnp.sum(partials[:, 16:32])
    val = jnp.sum(partials[:, 32:48])
    return a / (pos + EPS * val)


def kernel(output, target, labels):
    out_flat = jnp.reshape(output, (-1,))
    tgt_flat = jnp.reshape(target, (-1,))
    lbl_flat = jnp.reshape(labels, (-1,))
    return _rpn_loss(out_flat, tgt_flat, lbl_flat)


# speedup vs baseline: 39.7216x; 39.7216x over previous
import jax
import jax.numpy as jnp
from jax import lax
from jax.experimental import pallas as pl
from jax.experimental.pallas import tpu as pltpu
from jax.experimental.pallas import tpu_sc as plsc

A = 589824
EPS = 1e-7
NC = 2
NS = 16
NW = NC * NS
PER_W = A // NW
CHUNK = 4608
NCHUNK = PER_W // CHUNK
GROUPS = CHUNK // 16


def _sc_body(out_hbm, tgt_hbm, lbl_hbm, res_hbm,
             obuf0, tbuf0, lbuf0, obuf1, tbuf1, lbuf1, res_v,
             sem0, sem1):
    wid = lax.axis_index("s") * NC + lax.axis_index("c")
    a_base = wid * PER_W
    bufs = ((obuf0, tbuf0, lbuf0, sem0), (obuf1, tbuf1, lbuf1, sem1))

    def issue(c):
        ob, tb, lb, sem = bufs[c % 2]
        astart = a_base + c * CHUNK
        return (
            pltpu.async_copy(out_hbm.at[pl.ds(astart * 4, CHUNK * 4)], ob, sem),
            pltpu.async_copy(tgt_hbm.at[pl.ds(astart * 4, CHUNK * 4)], tb, sem),
            pltpu.async_copy(lbl_hbm.at[pl.ds(astart, CHUNK)], lb, sem),
        )

    pending = [issue(0), issue(1)]

    zero = jnp.zeros((16,), jnp.float32)
    one = jnp.ones((16,), jnp.float32)
    a_acc, p_acc, v_acc = zero, zero, zero

    for c in range(NCHUNK):
        for d in pending[c]:
            d.wait()
        ob, tb, lb, _ = bufs[c % 2]

        def jbody(j, carry, ob=ob, tb=tb, lb=lb):
            aa, pa, va = carry
            base = ((j >> 3) << 9) + ((j & 7) << 4)
            lbl16 = lb[pl.ds(j * 16, 16)]
            m16 = jnp.where(lbl16 > 0.0, one, zero)
            pa = pa + m16
            va = va + jnp.where(lbl16 != -1.0, one, zero)
            s = zero
            for k in range(4):
                o = ob[pl.ds(base + k * 128, 16)]
                t = tb[pl.ds(base + k * 128, 16)]
                diff = jnp.abs(o - t)
                s = s + jnp.where(diff < 1.0, 0.5 * diff * diff, diff - 0.5)
            aa = aa + m16 * s
            return aa, pa, va

        a_acc, p_acc, v_acc = lax.fori_loop(
            0, GROUPS, jbody, (a_acc, p_acc, v_acc))

        nxt = c + 2
        if nxt < NCHUNK:
            pending.append(issue(nxt))

    res_v[pl.ds(0, 16)] = a_acc
    res_v[pl.ds(16, 16)] = p_acc
    res_v[pl.ds(32, 16)] = v_acc
    pltpu.sync_copy(res_v, res_hbm.at[wid])


@jax.jit
def _rpn_loss(out_flat, tgt_flat, lbl_flat):
    mesh = plsc.VectorSubcoreMesh(core_axis_name="c", subcore_axis_name="s")
    partials = pl.kernel(
        _sc_body,
        mesh=mesh,
        out_type=jax.ShapeDtypeStruct((NW, 48), jnp.float32),
        scratch_types=[
            pltpu.VMEM((CHUNK * 4,), jnp.float32),
            pltpu.VMEM((CHUNK * 4,), jnp.float32),
            pltpu.VMEM((CHUNK,), jnp.float32),
            pltpu.VMEM((CHUNK * 4,), jnp.float32),
            pltpu.VMEM((CHUNK * 4,), jnp.float32),
            pltpu.VMEM((CHUNK,), jnp.float32),
            pltpu.VMEM((48,), jnp.float32),
            pltpu.SemaphoreType.DMA,
            pltpu.SemaphoreType.DMA,
        ],
    )(out_flat, tgt_flat, lbl_flat)
    a = jnp.sum(partials[:, 0:16])
    pos = jnp.sum(partials[:, 16:32])
    val = jnp.sum(partials[:, 32:48])
    return a / (pos + EPS * val)


def kernel(output, target, labels):
    out_flat = jnp.reshape(
        jnp.transpose(jnp.reshape(output, (A // 128, 128, 4)), (0, 2, 1)),
        (-1,))
    tgt_flat = jnp.reshape(
        jnp.transpose(jnp.reshape(target, (A // 128, 128, 4)), (0, 2, 1)),
        (-1,))
    lbl_flat = jnp.reshape(labels, (-1,))
    return _rpn_loss(out_flat, tgt_flat, lbl_flat)
